# Initial kernel scaffold; baseline (speedup 1.0000x reference)
import jax
import jax.numpy as jnp
from jax.experimental import pallas as pl


def kernel(x, edge_index, W1_l, b1_l, W1_r, W2_l, b2_l, W2_r):
    raise NotImplementedError("write your pallas kernel here")


# speedup vs baseline: 6.3670x; 6.3670x over previous
import jax
import jax.numpy as jnp
import numpy as np
from jax import lax
from jax.experimental import pallas as pl
from jax.experimental.pallas import tpu as pltpu
from jax.experimental.pallas import tpu_sc as plsc

N_NODES = 10000
N_EDGES = 320000
D = 128
DH = D // 2

NC = 2
NS = 16
NW = NC * NS
CHUNK = 128
CPW = 80
E_PAD = NW * CPW * CHUNK
N_ACC = N_NODES + 8
WB = 200
CP_TILES = 10
ROWS_PER_CP = N_NODES // CP_TILES
ZTILES = 8
ZROWS = N_ACC // ZTILES


def _make_sc_agg(with_counts: bool):
  mesh = plsc.VectorSubcoreMesh(core_axis_name="c", subcore_axis_name="s")
  out_type = [jax.ShapeDtypeStruct((NC, N_NODES, DH), jnp.float32)]
  scratch = [
      pltpu.VMEM((CPW, CHUNK), jnp.int32),
      pltpu.VMEM((CPW, CHUNK), jnp.int32),
      pltpu.VMEM((CHUNK, DH), jnp.float32),
      pltpu.VMEM((WB, DH), jnp.float32),
      pltpu.VMEM_SHARED((N_ACC, DH), jnp.float32),
      pltpu.SemaphoreType.DMA,
  ]
  if with_counts:
    out_type.append(jax.ShapeDtypeStruct((NC * N_NODES,), jnp.float32))
    scratch += [
        pltpu.VMEM((CHUNK,), jnp.float32),
        pltpu.VMEM((1008,), jnp.float32),
        pltpu.VMEM_SHARED((N_ACC,), jnp.float32),
    ]

  def body(*refs):
    if with_counts:
      (x_hbm, src_hbm, dst_hbm, out_hbm, cnt_hbm,
       srcv, dstv, msgs, wb, acc, sem, ones, zc, cacc) = refs
    else:
      (x_hbm, src_hbm, dst_hbm, out_hbm,
       srcv, dstv, msgs, wb, acc, sem) = refs

    c = lax.axis_index("c")
    s = lax.axis_index("s")
    wid = s * NC + c

    z16 = jnp.zeros((16,), jnp.float32)

    def zrow(r, carry):
      for k in range(DH // 16):
        wb[r, pl.ds(k * 16, 16)] = z16
      return carry

    lax.fori_loop(0, WB, zrow, 0)

    @pl.when(s < ZTILES)
    def _():
      off = 0
      while off < ZROWS:
        n = min(WB, ZROWS - off)
        pltpu.sync_copy(wb.at[pl.ds(0, n)],
                        acc.at[pl.ds(s * ZROWS + off, n)])
        off += n

    if with_counts:
      for k in range(CHUNK // 16):
        ones[pl.ds(k * 16, 16)] = jnp.ones((16,), jnp.float32)
      for k in range(63):
        zc[pl.ds(k * 16, 16)] = z16

      @pl.when(s < CP_TILES)
      def _():
        pltpu.sync_copy(zc.at[pl.ds(0, ROWS_PER_CP)],
                        cacc.at[pl.ds(s * ROWS_PER_CP, ROWS_PER_CP)])

    pltpu.sync_copy(src_hbm.at[wid], srcv)
    pltpu.sync_copy(dst_hbm.at[wid], dstv)
    plsc.subcore_barrier()

    def step(j, carry):
      pltpu.async_copy(x_hbm.at[srcv.at[j]], msgs, sem).wait()
      pltpu.sync_copy(msgs, acc.at[dstv.at[j]], add=True)
      if with_counts:
        pltpu.sync_copy(ones, cacc.at[dstv.at[j]], add=True)
      return carry

    lax.fori_loop(0, CPW, step, 0)
    plsc.subcore_barrier()

    @pl.when(s < CP_TILES)
    def _():
      for k in range(ROWS_PER_CP // WB):
        r0 = s * ROWS_PER_CP + k * WB
        pltpu.sync_copy(acc.at[pl.ds(r0, WB)], wb)
        pltpu.sync_copy(wb, out_hbm.at[c, pl.ds(r0, WB)])
      if with_counts:
        pltpu.sync_copy(cacc.at[pl.ds(s * ROWS_PER_CP, ROWS_PER_CP)],
                        zc.at[pl.ds(0, ROWS_PER_CP)])
        pltpu.sync_copy(
            zc.at[pl.ds(0, ROWS_PER_CP)],
            cnt_hbm.at[pl.ds(c * N_NODES + s * ROWS_PER_CP, ROWS_PER_CP)])

  return pl.kernel(
      body, out_type=out_type, mesh=mesh, scratch_types=scratch,
      compiler_params=pltpu.CompilerParams(use_tc_tiling_on_sc=False))


_sc_agg_counts = _make_sc_agg(True)
_sc_agg = _make_sc_agg(False)

_PAD = E_PAD - N_EDGES
_PAD_SRC = np.asarray((np.arange(_PAD) * 37) % N_NODES, dtype=np.int32)
_PAD_DST = np.asarray(N_NODES + (np.arange(_PAD) % 8), dtype=np.int32)

R = 1000


def _make_dense(split_out: bool):
  def dense_body(p0_ref, p1_ref, cnt_ref, x0_ref, x1_ref, wl_ref, b_ref,
                 wr_ref, *o_refs):
    cnt = jnp.maximum(cnt_ref[0] + cnt_ref[1], 1.0)
    agg = jnp.concatenate(
        [p0_ref[0] + p0_ref[1], p1_ref[0] + p1_ref[1]], axis=1) / cnt
    xx = jnp.concatenate([x0_ref[...], x1_ref[...]], axis=1)
    dn = (((1,), (1,)), ((), ()))
    h = lax.dot_general(agg, wl_ref[...], dn,
                        preferred_element_type=jnp.float32)
    h = h + lax.dot_general(xx, wr_ref[...], dn,
                            preferred_element_type=jnp.float32)
    h = jnp.maximum(h + b_ref[...], 0.0)
    if split_out:
      o_refs[0][...] = h[:, :DH]
      o_refs[1][...] = h[:, DH:]
    else:
      o_refs[0][...] = h

  if split_out:
    out_shape = [jax.ShapeDtypeStruct((N_NODES, DH), jnp.float32)] * 2
    out_specs = [pl.BlockSpec((R, DH), lambda i: (i, 0))] * 2
  else:
    out_shape = jax.ShapeDtypeStruct((N_NODES, D), jnp.float32)
    out_specs = pl.BlockSpec((R, D), lambda i: (i, 0))

  return pl.pallas_call(
      dense_body,
      grid=(N_NODES // R,),
      in_specs=[
          pl.BlockSpec((NC, R, DH), lambda i: (0, i, 0)),
          pl.BlockSpec((NC, R, DH), lambda i: (0, i, 0)),
          pl.BlockSpec((NC, R, 1), lambda i: (0, i, 0)),
          pl.BlockSpec((R, DH), lambda i: (i, 0)),
          pl.BlockSpec((R, DH), lambda i: (i, 0)),
          pl.BlockSpec((D, D), lambda i: (0, 0)),
          pl.BlockSpec((1, D), lambda i: (0, 0)),
          pl.BlockSpec((D, D), lambda i: (0, 0)),
      ],
      out_specs=out_specs,
      out_shape=out_shape,
  )


_dense_split = _make_dense(True)
_dense_full = _make_dense(False)


@jax.jit
def kernel(x, edge_index, W1_l, b1_l, W1_r, W2_l, b2_l, W2_r):
  src = jnp.concatenate(
      [edge_index[0].astype(jnp.int32), _PAD_SRC]).reshape(NW, CPW, CHUNK)
  dst = jnp.concatenate(
      [edge_index[1].astype(jnp.int32), _PAD_DST]).reshape(NW, CPW, CHUNK)
  x0 = x[:, :DH]
  x1 = x[:, DH:]

  p10, cnt = _sc_agg_counts(x0, src, dst)
  (p11,) = _sc_agg(x1, src, dst)
  cnt3 = cnt.reshape(NC, N_NODES, 1)
  h0, h1 = _dense_split(p10, p11, cnt3, x0, x1, W1_l,
                        b1_l.reshape(1, D), W1_r)
  (p20,) = _sc_agg(h0, src, dst)
  (p21,) = _sc_agg(h1, src, dst)
  return _dense_full(p20, p21, cnt3, h0, h1, W2_l, b2_l.reshape(1, D), W2_r)


# speedup vs baseline: 9.5423x; 1.4987x over previous
import jax
import jax.numpy as jnp
import numpy as np
from jax import lax
from jax.experimental import pallas as pl
from jax.experimental.pallas import tpu as pltpu
from jax.experimental.pallas import tpu_sc as plsc

N_NODES = 10000
N_EDGES = 320000
D = 128
DH = D // 2

NC = 2
NS = 16
NW = NC * NS
CHUNK = 128
CPW = 80
E_PAD = NW * CPW * CHUNK
N_ACC = N_NODES + 8
WB = 200
CP_TILES = 10
ROWS_PER_CP = N_NODES // CP_TILES
ZTILES = 8
ZROWS = N_ACC // ZTILES


def _make_sc_agg(with_counts: bool):
  mesh = plsc.VectorSubcoreMesh(core_axis_name="c", subcore_axis_name="s")
  out_type = [jax.ShapeDtypeStruct((NC, N_NODES, DH), jnp.float32)]
  scratch = [
      pltpu.VMEM((CPW, CHUNK), jnp.int32),
      pltpu.VMEM((CPW, CHUNK), jnp.int32),
      pltpu.VMEM((CHUNK, DH), jnp.float32),
      pltpu.VMEM((CHUNK, DH), jnp.float32),
      pltpu.VMEM((WB, DH), jnp.float32),
      pltpu.VMEM_SHARED((N_ACC, DH), jnp.float32),
      pltpu.SemaphoreType.DMA,
      pltpu.SemaphoreType.DMA,
  ]
  if with_counts:
    out_type.append(jax.ShapeDtypeStruct((NC * N_NODES,), jnp.float32))
    scratch += [
        pltpu.VMEM((CHUNK,), jnp.float32),
        pltpu.VMEM((1008,), jnp.float32),
        pltpu.VMEM_SHARED((N_ACC,), jnp.float32),
    ]

  def body(*refs):
    if with_counts:
      (x_hbm, src_hbm, dst_hbm, out_hbm, cnt_hbm,
       srcv, dstv, msgs0, msgs1, wb, acc, sem0, sem1,
       ones, zc, cacc) = refs
    else:
      (x_hbm, src_hbm, dst_hbm, out_hbm,
       srcv, dstv, msgs0, msgs1, wb, acc, sem0, sem1) = refs
    bufs = ((msgs0, sem0), (msgs1, sem1))

    c = lax.axis_index("c")
    s = lax.axis_index("s")
    wid = s * NC + c

    z16 = jnp.zeros((16,), jnp.float32)

    def zrow(r, carry):
      for k in range(DH // 16):
        wb[r, pl.ds(k * 16, 16)] = z16
      return carry

    lax.fori_loop(0, WB, zrow, 0)

    @pl.when(s < ZTILES)
    def _():
      off = 0
      while off < ZROWS:
        n = min(WB, ZROWS - off)
        pltpu.sync_copy(wb.at[pl.ds(0, n)],
                        acc.at[pl.ds(s * ZROWS + off, n)])
        off += n

    if with_counts:
      for k in range(CHUNK // 16):
        ones[pl.ds(k * 16, 16)] = jnp.ones((16,), jnp.float32)
      for k in range(63):
        zc[pl.ds(k * 16, 16)] = z16

      @pl.when(s < CP_TILES)
      def _():
        pltpu.sync_copy(zc.at[pl.ds(0, ROWS_PER_CP)],
                        cacc.at[pl.ds(s * ROWS_PER_CP, ROWS_PER_CP)])

    pltpu.sync_copy(src_hbm.at[wid], srcv)
    pltpu.sync_copy(dst_hbm.at[wid], dstv)
    plsc.subcore_barrier()

    for b, (msgs, sem) in enumerate(bufs):
      pltpu.async_copy(x_hbm.at[srcv.at[b]], msgs, sem)

    def step(i, carry):
      for b, (msgs, sem) in enumerate(bufs):
        j = 2 * i + b
        pltpu.make_async_copy(x_hbm.at[srcv.at[j]], msgs, sem).wait()
        pltpu.sync_copy(msgs, acc.at[dstv.at[j]], add=True)
        if with_counts:
          pltpu.sync_copy(ones, cacc.at[dstv.at[j]], add=True)
        nxt = j + 2

        @pl.when(nxt < CPW)
        def _():
          pltpu.async_copy(x_hbm.at[srcv.at[nxt]], msgs, sem)
      return carry

    lax.fori_loop(0, CPW // 2, step, 0)
    plsc.subcore_barrier()

    @pl.when(s < CP_TILES)
    def _():
      for k in range(ROWS_PER_CP // WB):
        r0 = s * ROWS_PER_CP + k * WB
        pltpu.sync_copy(acc.at[pl.ds(r0, WB)], wb)
        pltpu.sync_copy(wb, out_hbm.at[c, pl.ds(r0, WB)])
      if with_counts:
        pltpu.sync_copy(cacc.at[pl.ds(s * ROWS_PER_CP, ROWS_PER_CP)],
                        zc.at[pl.ds(0, ROWS_PER_CP)])
        pltpu.sync_copy(
            zc.at[pl.ds(0, ROWS_PER_CP)],
            cnt_hbm.at[pl.ds(c * N_NODES + s * ROWS_PER_CP, ROWS_PER_CP)])

  return pl.kernel(
      body, out_type=out_type, mesh=mesh, scratch_types=scratch,
      compiler_params=pltpu.CompilerParams(use_tc_tiling_on_sc=False))


_sc_agg_counts = _make_sc_agg(True)
_sc_agg = _make_sc_agg(False)

_PAD = E_PAD - N_EDGES
_PAD_SRC = np.asarray((np.arange(_PAD) * 37) % N_NODES, dtype=np.int32)
_PAD_DST = np.asarray(N_NODES + (np.arange(_PAD) % 8), dtype=np.int32)

R = 1000


def _make_dense(split_out: bool):
  def dense_body(p0_ref, p1_ref, cnt_ref, x0_ref, x1_ref, wl_ref, b_ref,
                 wr_ref, *o_refs):
    cnt = jnp.maximum(cnt_ref[0] + cnt_ref[1], 1.0)
    agg = jnp.concatenate(
        [p0_ref[0] + p0_ref[1], p1_ref[0] + p1_ref[1]], axis=1) / cnt
    xx = jnp.concatenate([x0_ref[...], x1_ref[...]], axis=1)
    dn = (((1,), (1,)), ((), ()))
    h = lax.dot_general(agg, wl_ref[...], dn,
                        preferred_element_type=jnp.float32)
    h = h + lax.dot_general(xx, wr_ref[...], dn,
                            preferred_element_type=jnp.float32)
    h = jnp.maximum(h + b_ref[...], 0.0)
    if split_out:
      o_refs[0][...] = h[:, :DH]
      o_refs[1][...] = h[:, DH:]
    else:
      o_refs[0][...] = h

  if split_out:
    out_shape = [jax.ShapeDtypeStruct((N_NODES, DH), jnp.float32)] * 2
    out_specs = [pl.BlockSpec((R, DH), lambda i: (i, 0))] * 2
  else:
    out_shape = jax.ShapeDtypeStruct((N_NODES, D), jnp.float32)
    out_specs = pl.BlockSpec((R, D), lambda i: (i, 0))

  return pl.pallas_call(
      dense_body,
      grid=(N_NODES // R,),
      in_specs=[
          pl.BlockSpec((NC, R, DH), lambda i: (0, i, 0)),
          pl.BlockSpec((NC, R, DH), lambda i: (0, i, 0)),
          pl.BlockSpec((NC, R, 1), lambda i: (0, i, 0)),
          pl.BlockSpec((R, DH), lambda i: (i, 0)),
          pl.BlockSpec((R, DH), lambda i: (i, 0)),
          pl.BlockSpec((D, D), lambda i: (0, 0)),
          pl.BlockSpec((1, D), lambda i: (0, 0)),
          pl.BlockSpec((D, D), lambda i: (0, 0)),
      ],
      out_specs=out_specs,
      out_shape=out_shape,
  )


_dense_split = _make_dense(True)
_dense_full = _make_dense(False)


@jax.jit
def kernel(x, edge_index, W1_l, b1_l, W1_r, W2_l, b2_l, W2_r):
  src = jnp.concatenate(
      [edge_index[0].astype(jnp.int32), _PAD_SRC]).reshape(NW, CPW, CHUNK)
  dst = jnp.concatenate(
      [edge_index[1].astype(jnp.int32), _PAD_DST]).reshape(NW, CPW, CHUNK)
  x0 = x[:, :DH]
  x1 = x[:, DH:]

  p10, cnt = _sc_agg_counts(x0, src, dst)
  (p11,) = _sc_agg(x1, src, dst)
  cnt3 = cnt.reshape(NC, N_NODES, 1)
  h0, h1 = _dense_split(p10, p11, cnt3, x0, x1, W1_l,
                        b1_l.reshape(1, D), W1_r)
  (p20,) = _sc_agg(h0, src, dst)
  (p21,) = _sc_agg(h1, src, dst)
  return _dense_full(p20, p21, cnt3, h0, h1, W2_l, b2_l.reshape(1, D), W2_r)


# speedup vs baseline: 11.7005x; 1.2262x over previous
import jax
import jax.numpy as jnp
import numpy as np
from jax import lax
from jax.experimental import pallas as pl
from jax.experimental.pallas import tpu as pltpu
from jax.experimental.pallas import tpu_sc as plsc

N_NODES = 10000
N_EDGES = 320000
D = 128
DH = D // 2

NC = 2
NS = 16
CHUNK = 128
CPT = 160
E_PAD = NS * CPT * CHUNK
N_ACC = N_NODES + 8
NBUF = 4
NR = CPT // NBUF
WB = 200
CP_TILES = 10
ROWS_PER_CP = N_NODES // CP_TILES
ZTILES = 8
ZROWS = N_ACC // ZTILES


def _make_sc_agg(with_counts: bool):
  mesh = plsc.VectorSubcoreMesh(core_axis_name="c", subcore_axis_name="s")
  out_type = [jax.ShapeDtypeStruct((NC, N_NODES, DH), jnp.float32)]
  scratch = [
      pltpu.VMEM((CPT, CHUNK), jnp.int32),
      pltpu.VMEM((CPT, CHUNK), jnp.int32),
      [pltpu.VMEM((CHUNK, DH), jnp.float32) for _ in range(NBUF)],
      pltpu.VMEM((WB, DH), jnp.float32),
      pltpu.VMEM_SHARED((N_ACC, DH), jnp.float32),
      [pltpu.SemaphoreType.DMA for _ in range(NBUF)],
      [pltpu.SemaphoreType.DMA for _ in range(NBUF)],
  ]
  if with_counts:
    out_type.append(jax.ShapeDtypeStruct((N_NODES,), jnp.float32))
    scratch += [
        pltpu.VMEM((CHUNK,), jnp.float32),
        pltpu.VMEM((1008,), jnp.float32),
        pltpu.VMEM_SHARED((N_ACC,), jnp.float32),
        pltpu.SemaphoreType.DMA,
    ]

  def body(*refs):
    if with_counts:
      (x0_hbm, x1_hbm, src_hbm, dst_hbm, out_hbm, cnt_hbm,
       srcv, dstv, msgs, wb, acc, gsem, ssem, ones, zc, cacc, csem) = refs
    else:
      (x0_hbm, x1_hbm, src_hbm, dst_hbm, out_hbm,
       srcv, dstv, msgs, wb, acc, gsem, ssem) = refs

    c = lax.axis_index("c")
    s = lax.axis_index("s")

    z16 = jnp.zeros((16,), jnp.float32)

    def zrow(r, carry):
      for k in range(DH // 16):
        wb[r, pl.ds(k * 16, 16)] = z16
      return carry

    lax.fori_loop(0, WB, zrow, 0)

    @pl.when(s < ZTILES)
    def _():
      off = 0
      while off < ZROWS:
        n = min(WB, ZROWS - off)
        pltpu.sync_copy(wb.at[pl.ds(0, n)],
                        acc.at[pl.ds(s * ZROWS + off, n)])
        off += n

    if with_counts:
      for k in range(CHUNK // 16):
        ones[pl.ds(k * 16, 16)] = jnp.ones((16,), jnp.float32)
      for k in range(63):
        zc[pl.ds(k * 16, 16)] = z16

      @pl.when((c == 0) & (s < CP_TILES))
      def _():
        pltpu.sync_copy(zc.at[pl.ds(0, ROWS_PER_CP)],
                        cacc.at[pl.ds(s * ROWS_PER_CP, ROWS_PER_CP)])

    pltpu.sync_copy(src_hbm.at[s], srcv)
    pltpu.sync_copy(dst_hbm.at[s], dstv)
    plsc.subcore_barrier()

    def run_half(x_hbm, do_counts):
      def step(i, carry):
        for b in range(NBUF):
          j = i * NBUF + b

          @pl.when(i > 0)
          def _():
            pltpu.make_async_copy(
                msgs[b], acc.at[dstv.at[j - NBUF]], ssem[b]).wait()

          pltpu.async_copy(x_hbm.at[srcv.at[j]], msgs[b], gsem[b])

        for b in range(NBUF):
          j = i * NBUF + b
          pltpu.make_async_copy(x_hbm.at[srcv.at[j]], msgs[b], gsem[b]).wait()
          pltpu.async_copy(msgs[b], acc.at[dstv.at[j]], ssem[b], add=True)
          if do_counts:
            pltpu.async_copy(ones, cacc.at[dstv.at[j]], csem, add=True)
        return carry

      lax.fori_loop(0, NR, step, 0)

      for b in range(NBUF):
        pltpu.make_async_copy(
            msgs[b], acc.at[dstv.at[CPT - NBUF + b]], ssem[b]).wait()
      if do_counts:
        def cdrain(j, carry):
          pltpu.make_async_copy(ones, cacc.at[dstv.at[j]], csem).wait()
          return carry
        lax.fori_loop(0, CPT, cdrain, 0)

    @pl.when(c == 0)
    def _():
      run_half(x0_hbm, with_counts)

    @pl.when(c == 1)
    def _():
      run_half(x1_hbm, False)

    plsc.subcore_barrier()

    @pl.when(s < CP_TILES)
    def _():
      for k in range(ROWS_PER_CP // WB):
        r0 = s * ROWS_PER_CP + k * WB
        pltpu.sync_copy(acc.at[pl.ds(r0, WB)], wb)
        pltpu.sync_copy(wb, out_hbm.at[c, pl.ds(r0, WB)])

    if with_counts:
      @pl.when((c == 0) & (s < CP_TILES))
      def _():
        pltpu.sync_copy(cacc.at[pl.ds(s * ROWS_PER_CP, ROWS_PER_CP)],
                        zc.at[pl.ds(0, ROWS_PER_CP)])
        pltpu.sync_copy(zc.at[pl.ds(0, ROWS_PER_CP)],
                        cnt_hbm.at[pl.ds(s * ROWS_PER_CP, ROWS_PER_CP)])

  return pl.kernel(
      body, out_type=out_type, mesh=mesh, scratch_types=scratch,
      compiler_params=pltpu.CompilerParams(use_tc_tiling_on_sc=False))


_sc_agg_counts = _make_sc_agg(True)
_sc_agg = _make_sc_agg(False)

_PAD = E_PAD - N_EDGES
_PAD_SRC = np.asarray((np.arange(_PAD) * 37) % N_NODES, dtype=np.int32)
_PAD_DST = np.asarray(N_NODES + (np.arange(_PAD) % 8), dtype=np.int32)

R = 1000


def _make_dense(split_out: bool):
  def dense_body(p_ref, cnt_ref, x0_ref, x1_ref, wl_ref, b_ref,
                 wr_ref, *o_refs):
    cnt = jnp.maximum(cnt_ref[...], 1.0)
    agg = jnp.concatenate([p_ref[0], p_ref[1]], axis=1) / cnt
    xx = jnp.concatenate([x0_ref[...], x1_ref[...]], axis=1)
    dn = (((1,), (1,)), ((), ()))
    h = lax.dot_general(agg, wl_ref[...], dn,
                        preferred_element_type=jnp.float32)
    h = h + lax.dot_general(xx, wr_ref[...], dn,
                            preferred_element_type=jnp.float32)
    h = jnp.maximum(h + b_ref[...], 0.0)
    if split_out:
      o_refs[0][...] = h[:, :DH]
      o_refs[1][...] = h[:, DH:]
    else:
      o_refs[0][...] = h

  if split_out:
    out_shape = [jax.ShapeDtypeStruct((N_NODES, DH), jnp.float32)] * 2
    out_specs = [pl.BlockSpec((R, DH), lambda i: (i, 0))] * 2
  else:
    out_shape = jax.ShapeDtypeStruct((N_NODES, D), jnp.float32)
    out_specs = pl.BlockSpec((R, D), lambda i: (i, 0))

  return pl.pallas_call(
      dense_body,
      grid=(N_NODES // R,),
      in_specs=[
          pl.BlockSpec((NC, R, DH), lambda i: (0, i, 0)),
          pl.BlockSpec((R, 1), lambda i: (i, 0)),
          pl.BlockSpec((R, DH), lambda i: (i, 0)),
          pl.BlockSpec((R, DH), lambda i: (i, 0)),
          pl.BlockSpec((D, D), lambda i: (0, 0)),
          pl.BlockSpec((1, D), lambda i: (0, 0)),
          pl.BlockSpec((D, D), lambda i: (0, 0)),
      ],
      out_specs=out_specs,
      out_shape=out_shape,
  )


_dense_split = _make_dense(True)
_dense_full = _make_dense(False)


@jax.jit
def kernel(x, edge_index, W1_l, b1_l, W1_r, W2_l, b2_l, W2_r):
  src = jnp.concatenate(
      [edge_index[0].astype(jnp.int32), _PAD_SRC]).reshape(NS, CPT, CHUNK)
  dst = jnp.concatenate(
      [edge_index[1].astype(jnp.int32), _PAD_DST]).reshape(NS, CPT, CHUNK)
  x0 = x[:, :DH]
  x1 = x[:, DH:]

  p1, cnt = _sc_agg_counts(x0, x1, src, dst)
  cnt2 = cnt.reshape(N_NODES, 1)
  h0, h1 = _dense_split(p1, cnt2, x0, x1, W1_l, b1_l.reshape(1, D), W1_r)
  (p2,) = _sc_agg(h0, h1, src, dst)
  return _dense_full(p2, cnt2, h0, h1, W2_l, b2_l.reshape(1, D), W2_r)


# speedup vs baseline: 13.0764x; 1.1176x over previous
import jax
import jax.numpy as jnp
from jax import lax
from jax.experimental import pallas as pl
from jax.experimental.pallas import tpu as pltpu
from jax.experimental.pallas import tpu_sc as plsc

N_NODES = 10000
N_EDGES = 320000
D = 128
DH = D // 2

NC = 2
NS = 16
CHUNK = 125
CPT = 160
NBUF = 4
NR = CPT // NBUF
WB = 200
CP_TILES = 10
ROWS_PER_CP = N_NODES // CP_TILES
ZTILES = 8
ZROWS = N_NODES // ZTILES


def _make_sc_agg(with_counts: bool):
  mesh = plsc.VectorSubcoreMesh(core_axis_name="c", subcore_axis_name="s")
  out_type = [jax.ShapeDtypeStruct((N_NODES, D), jnp.float32)]
  scratch = [
      pltpu.VMEM((CPT, CHUNK), jnp.int32),
      pltpu.VMEM((CPT, CHUNK), jnp.int32),
      [pltpu.VMEM((CHUNK, DH), jnp.float32) for _ in range(NBUF)],
      pltpu.VMEM((WB, DH), jnp.float32),
      pltpu.VMEM_SHARED((N_NODES, DH), jnp.float32),
      [pltpu.SemaphoreType.DMA for _ in range(NBUF)],
      [pltpu.SemaphoreType.DMA for _ in range(NBUF)],
  ]
  if with_counts:
    out_type.append(jax.ShapeDtypeStruct((N_NODES,), jnp.float32))
    scratch += [
        pltpu.VMEM((128,), jnp.float32),
        pltpu.VMEM((1008,), jnp.float32),
        pltpu.VMEM_SHARED((N_NODES,), jnp.float32),
        pltpu.SemaphoreType.DMA,
    ]

  def body(*refs):
    if with_counts:
      (xr_hbm, src_hbm, dst_hbm, out_hbm, cnt_hbm,
       srcv, dstv, msgs, wb, acc, gsem, ssem, ones, zc, cacc, csem) = refs
    else:
      (xr_hbm, src_hbm, dst_hbm, out_hbm,
       srcv, dstv, msgs, wb, acc, gsem, ssem) = refs

    c = lax.axis_index("c")
    s = lax.axis_index("s")

    z16 = jnp.zeros((16,), jnp.float32)

    def zrow(r, carry):
      for k in range(DH // 16):
        wb[r, pl.ds(k * 16, 16)] = z16
      return carry

    lax.fori_loop(0, WB, zrow, 0)

    @pl.when(s < ZTILES)
    def _():
      off = 0
      while off < ZROWS:
        n = min(WB, ZROWS - off)
        pltpu.sync_copy(wb.at[pl.ds(0, n)],
                        acc.at[pl.ds(s * ZROWS + off, n)])
        off += n

    if with_counts:
      for k in range(8):
        ones[pl.ds(k * 16, 16)] = jnp.ones((16,), jnp.float32)
      for k in range(63):
        zc[pl.ds(k * 16, 16)] = z16

      @pl.when((c == 0) & (s < CP_TILES))
      def _():
        pltpu.sync_copy(zc.at[pl.ds(0, ROWS_PER_CP)],
                        cacc.at[pl.ds(s * ROWS_PER_CP, ROWS_PER_CP)])

    pltpu.sync_copy(src_hbm.at[c, s], srcv)
    pltpu.sync_copy(dst_hbm.at[s], dstv)
    plsc.subcore_barrier()

    do_counts = with_counts

    def step(i, carry):
      for b in range(NBUF):
        j = i * NBUF + b

        @pl.when(i > 0)
        def _():
          pltpu.make_async_copy(
              msgs[b], acc.at[dstv.at[j - NBUF]], ssem[b]).wait()

        pltpu.async_copy(xr_hbm.at[srcv.at[j]], msgs[b], gsem[b])

      for b in range(NBUF):
        j = i * NBUF + b
        pltpu.make_async_copy(xr_hbm.at[srcv.at[j]], msgs[b], gsem[b]).wait()
        pltpu.async_copy(msgs[b], acc.at[dstv.at[j]], ssem[b], add=True)
        if do_counts:
          @pl.when(c == 0)
          def _():
            pltpu.async_copy(ones.at[pl.ds(0, CHUNK)], cacc.at[dstv.at[j]],
                             csem, add=True)
      return carry

    lax.fori_loop(0, NR, step, 0)

    for b in range(NBUF):
      pltpu.make_async_copy(
          msgs[b], acc.at[dstv.at[CPT - NBUF + b]], ssem[b]).wait()
    if do_counts:
      @pl.when(c == 0)
      def _():
        def cdrain(j, carry):
          pltpu.make_async_copy(ones.at[pl.ds(0, CHUNK)],
                                cacc.at[dstv.at[j]], csem).wait()
          return carry
        lax.fori_loop(0, CPT, cdrain, 0)

    plsc.subcore_barrier()

    @pl.when(s < CP_TILES)
    def _():
      for k in range(ROWS_PER_CP // WB):
        r0 = s * ROWS_PER_CP + k * WB
        pltpu.sync_copy(acc.at[pl.ds(r0, WB)], wb)
        pltpu.sync_copy(wb, out_hbm.at[pl.ds(r0, WB), pl.ds(c * DH, DH)])

    if with_counts:
      @pl.when((c == 0) & (s < CP_TILES))
      def _():
        pltpu.sync_copy(cacc.at[pl.ds(s * ROWS_PER_CP, ROWS_PER_CP)],
                        zc.at[pl.ds(0, ROWS_PER_CP)])
        pltpu.sync_copy(zc.at[pl.ds(0, ROWS_PER_CP)],
                        cnt_hbm.at[pl.ds(s * ROWS_PER_CP, ROWS_PER_CP)])

  return pl.kernel(
      body, out_type=out_type, mesh=mesh, scratch_types=scratch,
      compiler_params=pltpu.CompilerParams(use_tc_tiling_on_sc=False))


_sc_agg_counts = _make_sc_agg(True)
_sc_agg = _make_sc_agg(False)

R = 1000


def _dense_body(p_ref, cnt_ref, x_ref, wl_ref, b_ref, wr_ref, o_ref):
  cnt = jnp.maximum(cnt_ref[...], 1.0)
  agg = p_ref[...] / cnt
  dn = (((1,), (1,)), ((), ()))
  h = lax.dot_general(agg, wl_ref[...], dn, preferred_element_type=jnp.float32)
  h = h + lax.dot_general(x_ref[...], wr_ref[...], dn,
                          preferred_element_type=jnp.float32)
  o_ref[...] = jnp.maximum(h + b_ref[...], 0.0)


_dense = pl.pallas_call(
    _dense_body,
    grid=(N_NODES // R,),
    in_specs=[
        pl.BlockSpec((R, D), lambda i: (i, 0)),
        pl.BlockSpec((R, 1), lambda i: (i, 0)),
        pl.BlockSpec((R, D), lambda i: (i, 0)),
        pl.BlockSpec((D, D), lambda i: (0, 0)),
        pl.BlockSpec((1, D), lambda i: (0, 0)),
        pl.BlockSpec((D, D), lambda i: (0, 0)),
    ],
    out_specs=pl.BlockSpec((R, D), lambda i: (i, 0)),
    out_shape=jax.ShapeDtypeStruct((N_NODES, D), jnp.float32),
)


@jax.jit
def kernel(x, edge_index, W1_l, b1_l, W1_r, W2_l, b2_l, W2_r):
  s2 = edge_index[0].astype(jnp.int32) * 2
  src2 = jnp.stack([s2, s2 + 1]).reshape(NC, NS, CPT, CHUNK)
  dst = edge_index[1].astype(jnp.int32).reshape(NS, CPT, CHUNK)
  xr = x.reshape(2 * N_NODES, DH)

  p1, cnt = _sc_agg_counts(xr, src2, dst)
  cnt2 = cnt.reshape(N_NODES, 1)
  h = _dense(p1, cnt2, x, W1_l, b1_l.reshape(1, D), W1_r)
  (p2,) = _sc_agg(h.reshape(2 * N_NODES, DH), src2, dst)
  return _dense(p2, cnt2, h, W2_l, b2_l.reshape(1, D), W2_r)


# speedup vs baseline: 13.4106x; 1.0256x over previous
import jax
import jax.numpy as jnp
from jax import lax
from jax.experimental import pallas as pl
from jax.experimental.pallas import tpu as pltpu
from jax.experimental.pallas import tpu_sc as plsc

N_NODES = 10000
N_EDGES = 320000
D = 128
DH = D // 2

NC = 2
NS = 16
CHUNK = 125
CPT = 160
NBUF = 4
NR = CPT // NBUF
WB = 200
CP_TILES = 10
ROWS_PER_CP = N_NODES // CP_TILES
ZTILES = 8
ZROWS = N_NODES // ZTILES


def _make_sc_agg(with_counts: bool):
  mesh = plsc.VectorSubcoreMesh(core_axis_name="c", subcore_axis_name="s")
  out_type = [jax.ShapeDtypeStruct((N_NODES, D), jnp.float32)]
  scratch = [
      pltpu.VMEM((CPT, CHUNK), jnp.int32),
      pltpu.VMEM((CPT, CHUNK), jnp.int32),
      [pltpu.VMEM((CHUNK, DH), jnp.float32) for _ in range(NBUF)],
      pltpu.VMEM((WB, DH), jnp.float32),
      pltpu.VMEM_SHARED((N_NODES, DH), jnp.float32),
      [pltpu.SemaphoreType.DMA for _ in range(NBUF)],
      [pltpu.SemaphoreType.DMA for _ in range(NBUF)],
  ]
  if with_counts:
    out_type.append(jax.ShapeDtypeStruct((N_NODES,), jnp.float32))
    scratch += [
        pltpu.VMEM((128,), jnp.float32),
        pltpu.VMEM((1008,), jnp.float32),
        pltpu.VMEM_SHARED((N_NODES,), jnp.float32),
        pltpu.SemaphoreType.DMA,
    ]

  def body(*refs):
    if with_counts:
      (xr_hbm, idx_hbm, out_hbm, cnt_hbm,
       srcv, dstv, msgs, wb, acc, gsem, ssem, ones, zc, cacc, csem) = refs
    else:
      (xr_hbm, idx_hbm, out_hbm,
       srcv, dstv, msgs, wb, acc, gsem, ssem) = refs

    c = lax.axis_index("c")
    s = lax.axis_index("s")

    z16 = jnp.zeros((16,), jnp.float32)

    def zrow(r, carry):
      for k in range(DH // 16):
        wb[r, pl.ds(k * 16, 16)] = z16
      return carry

    lax.fori_loop(0, WB, zrow, 0)

    @pl.when(s < ZTILES)
    def _():
      off = 0
      while off < ZROWS:
        n = min(WB, ZROWS - off)
        pltpu.sync_copy(wb.at[pl.ds(0, n)],
                        acc.at[pl.ds(s * ZROWS + off, n)])
        off += n

    if with_counts:
      for k in range(8):
        ones[pl.ds(k * 16, 16)] = jnp.ones((16,), jnp.float32)
      for k in range(63):
        zc[pl.ds(k * 16, 16)] = z16

      @pl.when((c == 0) & (s < CP_TILES))
      def _():
        pltpu.sync_copy(zc.at[pl.ds(0, ROWS_PER_CP)],
                        cacc.at[pl.ds(s * ROWS_PER_CP, ROWS_PER_CP)])

    pltpu.sync_copy(idx_hbm.at[c, s], srcv)
    pltpu.sync_copy(idx_hbm.at[2, s], dstv)
    plsc.subcore_barrier()

    do_counts = with_counts

    def step(i, carry):
      for b in range(NBUF):
        j = i * NBUF + b

        @pl.when(i > 0)
        def _():
          pltpu.make_async_copy(
              msgs[b], acc.at[dstv.at[j - NBUF]], ssem[b]).wait()

        pltpu.async_copy(xr_hbm.at[srcv.at[j]], msgs[b], gsem[b])

      for b in range(NBUF):
        j = i * NBUF + b
        pltpu.make_async_copy(xr_hbm.at[srcv.at[j]], msgs[b], gsem[b]).wait()
        pltpu.async_copy(msgs[b], acc.at[dstv.at[j]], ssem[b], add=True)
        if do_counts:
          @pl.when(c == 0)
          def _():
            pltpu.async_copy(ones.at[pl.ds(0, CHUNK)], cacc.at[dstv.at[j]],
                             csem, add=True)
      return carry

    lax.fori_loop(0, NR, step, 0)

    for b in range(NBUF):
      pltpu.make_async_copy(
          msgs[b], acc.at[dstv.at[CPT - NBUF + b]], ssem[b]).wait()
    if do_counts:
      @pl.when(c == 0)
      def _():
        def cdrain(j, carry):
          pltpu.make_async_copy(ones.at[pl.ds(0, CHUNK)],
                                cacc.at[dstv.at[j]], csem).wait()
          return carry
        lax.fori_loop(0, CPT, cdrain, 0)

    plsc.subcore_barrier()

    @pl.when(s < CP_TILES)
    def _():
      for k in range(ROWS_PER_CP // WB):
        r0 = s * ROWS_PER_CP + k * WB
        pltpu.sync_copy(acc.at[pl.ds(r0, WB)], wb)
        pltpu.sync_copy(wb, out_hbm.at[pl.ds(r0, WB), pl.ds(c * DH, DH)])

    if with_counts:
      @pl.when((c == 0) & (s < CP_TILES))
      def _():
        pltpu.sync_copy(cacc.at[pl.ds(s * ROWS_PER_CP, ROWS_PER_CP)],
                        zc.at[pl.ds(0, ROWS_PER_CP)])
        pltpu.sync_copy(zc.at[pl.ds(0, ROWS_PER_CP)],
                        cnt_hbm.at[pl.ds(s * ROWS_PER_CP, ROWS_PER_CP)])

  return pl.kernel(
      body, out_type=out_type, mesh=mesh, scratch_types=scratch,
      compiler_params=pltpu.CompilerParams(use_tc_tiling_on_sc=False))


_sc_agg_counts = _make_sc_agg(True)
_sc_agg = _make_sc_agg(False)

R = 2000


def _dense_body(p_ref, cnt_ref, x_ref, wl_ref, b_ref, wr_ref, o_ref):
  cnt = jnp.maximum(cnt_ref[...], 1.0)
  agg = p_ref[...] / cnt
  dn = (((1,), (1,)), ((), ()))
  h = lax.dot_general(agg, wl_ref[...], dn, preferred_element_type=jnp.float32)
  h = h + lax.dot_general(x_ref[...], wr_ref[...], dn,
                          preferred_element_type=jnp.float32)
  o_ref[...] = jnp.maximum(h + b_ref[...], 0.0)


_dense = pl.pallas_call(
    _dense_body,
    grid=(N_NODES // R,),
    in_specs=[
        pl.BlockSpec((R, D), lambda i: (i, 0)),
        pl.BlockSpec((R, 1), lambda i: (i, 0)),
        pl.BlockSpec((R, D), lambda i: (i, 0)),
        pl.BlockSpec((D, D), lambda i: (0, 0)),
        pl.BlockSpec((1, D), lambda i: (0, 0)),
        pl.BlockSpec((D, D), lambda i: (0, 0)),
    ],
    out_specs=pl.BlockSpec((R, D), lambda i: (i, 0)),
    out_shape=jax.ShapeDtypeStruct((N_NODES, D), jnp.float32),
)


@jax.jit
def kernel(x, edge_index, W1_l, b1_l, W1_r, W2_l, b2_l, W2_r):
  ei = edge_index.astype(jnp.int32)
  s2 = ei[0:1] * 2
  idx = jnp.concatenate([s2, s2 + 1, ei[1:2]], axis=0).reshape(
      3, NS, CPT, CHUNK)
  xr = x.reshape(2 * N_NODES, DH)

  p1, cnt = _sc_agg_counts(xr, idx)
  cnt2 = cnt.reshape(N_NODES, 1)
  h = _dense(p1, cnt2, x, W1_l, b1_l.reshape(1, D), W1_r)
  (p2,) = _sc_agg(h.reshape(2 * N_NODES, DH), idx)
  return _dense(p2, cnt2, h, W2_l, b2_l.reshape(1, D), W2_r)


# speedup vs baseline: 13.6868x; 1.0206x over previous
import jax
import jax.numpy as jnp
from jax import lax
from jax.experimental import pallas as pl
from jax.experimental.pallas import tpu as pltpu
from jax.experimental.pallas import tpu_sc as plsc

N_NODES = 10000
N_EDGES = 320000
D = 128
DH = D // 2

NC = 2
NS = 16
CHUNK = 125
CPT = 160
NBUF = 4
NR = CPT // NBUF
WB = 200
CP_TILES = 10
ROWS_PER_CP = N_NODES // CP_TILES
ZTILES = 8
ZROWS = N_NODES // ZTILES


def _make_sc_agg(with_counts: bool):
  mesh = plsc.VectorSubcoreMesh(core_axis_name="c", subcore_axis_name="s")
  out_type = [jax.ShapeDtypeStruct((N_NODES, D), jnp.float32)]
  scratch = [
      pltpu.VMEM((CPT, CHUNK), jnp.int32),
      pltpu.VMEM((CPT, CHUNK), jnp.int32),
      [pltpu.VMEM((CHUNK, DH), jnp.float32) for _ in range(NBUF)],
      pltpu.VMEM((WB, DH), jnp.float32),
      pltpu.VMEM_SHARED((N_NODES, DH), jnp.float32),
      [pltpu.SemaphoreType.DMA for _ in range(NBUF)],
      [pltpu.SemaphoreType.DMA for _ in range(NBUF)],
  ]
  if with_counts:
    out_type.append(jax.ShapeDtypeStruct((N_NODES,), jnp.float32))
    scratch += [
        pltpu.VMEM((128,), jnp.float32),
        pltpu.VMEM((1008,), jnp.float32),
        pltpu.VMEM_SHARED((N_NODES,), jnp.float32),
        pltpu.SemaphoreType.DMA,
    ]

  def body(*refs):
    if with_counts:
      (xr_hbm, idx_hbm, out_hbm, cnt_hbm,
       srcv, dstv, msgs, wb, acc, gsem, ssem, ones, zc, cacc, csem) = refs
    else:
      (xr_hbm, idx_hbm, out_hbm,
       srcv, dstv, msgs, wb, acc, gsem, ssem) = refs

    c = lax.axis_index("c")
    s = lax.axis_index("s")

    pltpu.async_copy(idx_hbm.at[c, s], srcv, gsem[0])
    pltpu.async_copy(idx_hbm.at[2, s], dstv, gsem[1])

    z16 = jnp.zeros((16,), jnp.float32)

    def zrow(r, carry):
      for k in range(DH // 16):
        wb[r, pl.ds(k * 16, 16)] = z16
      return carry

    lax.fori_loop(0, WB, zrow, 0)

    @pl.when(s < ZTILES)
    def _():
      off = 0
      while off < ZROWS:
        n = min(WB, ZROWS - off)
        pltpu.sync_copy(wb.at[pl.ds(0, n)],
                        acc.at[pl.ds(s * ZROWS + off, n)])
        off += n

    if with_counts:
      for k in range(8):
        ones[pl.ds(k * 16, 16)] = jnp.ones((16,), jnp.float32)
      for k in range(63):
        zc[pl.ds(k * 16, 16)] = z16

      @pl.when((c == 0) & (s < CP_TILES))
      def _():
        pltpu.sync_copy(zc.at[pl.ds(0, ROWS_PER_CP)],
                        cacc.at[pl.ds(s * ROWS_PER_CP, ROWS_PER_CP)])

    pltpu.make_async_copy(idx_hbm.at[c, s], srcv, gsem[0]).wait()
    pltpu.make_async_copy(idx_hbm.at[2, s], dstv, gsem[1]).wait()
    plsc.subcore_barrier()

    do_counts = with_counts

    def step(i, carry):
      for b in range(NBUF):
        j = i * NBUF + b

        @pl.when(i > 0)
        def _():
          pltpu.make_async_copy(
              msgs[b], acc.at[dstv.at[j - NBUF]], ssem[b]).wait()

        pltpu.async_copy(xr_hbm.at[srcv.at[j]], msgs[b], gsem[b])

      for b in range(NBUF):
        j = i * NBUF + b
        pltpu.make_async_copy(xr_hbm.at[srcv.at[j]], msgs[b], gsem[b]).wait()
        pltpu.async_copy(msgs[b], acc.at[dstv.at[j]], ssem[b], add=True)
        if do_counts:
          @pl.when(c == 0)
          def _():
            pltpu.async_copy(ones.at[pl.ds(0, CHUNK)], cacc.at[dstv.at[j]],
                             csem, add=True)
      return carry

    lax.fori_loop(0, NR, step, 0)

    for b in range(NBUF):
      pltpu.make_async_copy(
          msgs[b], acc.at[dstv.at[CPT - NBUF + b]], ssem[b]).wait()
    if do_counts:
      @pl.when(c == 0)
      def _():
        def cdrain(j, carry):
          pltpu.make_async_copy(ones.at[pl.ds(0, CHUNK)],
                                cacc.at[dstv.at[j]], csem).wait()
          return carry
        lax.fori_loop(0, CPT, cdrain, 0)

    plsc.subcore_barrier()

    @pl.when(s < CP_TILES)
    def _():
      for k in range(ROWS_PER_CP // WB):
        r0 = s * ROWS_PER_CP + k * WB
        pltpu.sync_copy(acc.at[pl.ds(r0, WB)], wb)
        pltpu.sync_copy(wb, out_hbm.at[pl.ds(r0, WB), pl.ds(c * DH, DH)])

    if with_counts:
      @pl.when((c == 0) & (s < CP_TILES))
      def _():
        pltpu.sync_copy(cacc.at[pl.ds(s * ROWS_PER_CP, ROWS_PER_CP)],
                        zc.at[pl.ds(0, ROWS_PER_CP)])
        pltpu.sync_copy(zc.at[pl.ds(0, ROWS_PER_CP)],
                        cnt_hbm.at[pl.ds(s * ROWS_PER_CP, ROWS_PER_CP)])

  return pl.kernel(
      body, out_type=out_type, mesh=mesh, scratch_types=scratch,
      compiler_params=pltpu.CompilerParams(use_tc_tiling_on_sc=False))


_sc_agg_counts = _make_sc_agg(True)
_sc_agg = _make_sc_agg(False)

def _dense_body(p_ref, cnt_ref, x_ref, wl_ref, b_ref, wr_ref, o_ref):
  cnt = jnp.maximum(cnt_ref[...], 1.0)
  dn0 = (((0,), (0,)), ((), ()))
  cnt_col = lax.dot_general(cnt, jnp.ones((1, 1), jnp.float32), dn0,
                            preferred_element_type=jnp.float32,
                            precision=lax.Precision.HIGHEST)
  agg = p_ref[...] / cnt_col
  dn = (((1,), (1,)), ((), ()))
  h = lax.dot_general(agg, wl_ref[...], dn, preferred_element_type=jnp.float32)
  h = h + lax.dot_general(x_ref[...], wr_ref[...], dn,
                          preferred_element_type=jnp.float32)
  o_ref[...] = jnp.maximum(h + b_ref[...], 0.0)


_dense = pl.pallas_call(
    _dense_body,
    in_specs=[
        pl.BlockSpec((N_NODES, D), lambda: (0, 0)),
        pl.BlockSpec((1, N_NODES), lambda: (0, 0)),
        pl.BlockSpec((N_NODES, D), lambda: (0, 0)),
        pl.BlockSpec((D, D), lambda: (0, 0)),
        pl.BlockSpec((1, D), lambda: (0, 0)),
        pl.BlockSpec((D, D), lambda: (0, 0)),
    ],
    out_specs=pl.BlockSpec((N_NODES, D), lambda: (0, 0)),
    out_shape=jax.ShapeDtypeStruct((N_NODES, D), jnp.float32),
)


@jax.jit
def kernel(x, edge_index, W1_l, b1_l, W1_r, W2_l, b2_l, W2_r):
  ei = edge_index.astype(jnp.int32)
  s2 = ei[0:1] * 2
  idx = jnp.concatenate([s2, s2 + 1, ei[1:2]], axis=0).reshape(
      3, NS, CPT, CHUNK)
  xr = x.reshape(2 * N_NODES, DH)

  p1, cnt = _sc_agg_counts(xr, idx)
  cnt2 = cnt.reshape(1, N_NODES)
  h = _dense(p1, cnt2, x, W1_l, b1_l.reshape(1, D), W1_r)
  (p2,) = _sc_agg(h.reshape(2 * N_NODES, DH), idx)
  return _dense(p2, cnt2, h, W2_l, b2_l.reshape(1, D), W2_r)


# speedup vs baseline: 14.0187x; 1.0242x over previous
import jax
import jax.numpy as jnp
from jax import lax
from jax.experimental import pallas as pl
from jax.experimental.pallas import tpu as pltpu
from jax.experimental.pallas import tpu_sc as plsc

N_NODES = 10000
N_EDGES = 320000
D = 128
DH = D // 2

NC = 2
NS = 16
CHUNK = 125
CPT = 160
NBUF = 4
NR = CPT // NBUF
WB = 200
CP_TILES = 10
ROWS_PER_CP = N_NODES // CP_TILES
ZROWS = N_NODES // NS
WROWS = N_NODES // NS


def _make_sc_agg(with_counts: bool):
  mesh = plsc.VectorSubcoreMesh(core_axis_name="c", subcore_axis_name="s")
  out_type = [jax.ShapeDtypeStruct((N_NODES, D), jnp.float32)]
  scratch = [
      pltpu.VMEM((CPT, CHUNK), jnp.int32),
      pltpu.VMEM((CPT, CHUNK), jnp.int32),
      [pltpu.VMEM((CHUNK, DH), jnp.float32) for _ in range(NBUF)],
      pltpu.VMEM((WB, DH), jnp.float32),
      pltpu.VMEM_SHARED((N_NODES, DH), jnp.float32),
      [pltpu.SemaphoreType.DMA for _ in range(NBUF)],
      [pltpu.SemaphoreType.DMA for _ in range(NBUF)],
  ]
  if with_counts:
    out_type.append(jax.ShapeDtypeStruct((N_NODES,), jnp.float32))
    scratch += [
        pltpu.VMEM((128,), jnp.float32),
        pltpu.VMEM((1008,), jnp.float32),
        pltpu.VMEM_SHARED((N_NODES,), jnp.float32),
        pltpu.SemaphoreType.DMA,
    ]

  def body(*refs):
    if with_counts:
      (xr_hbm, idx_hbm, out_hbm, cnt_hbm,
       srcv, dstv, msgs, wb, acc, gsem, ssem, ones, zc, cacc, csem) = refs
    else:
      (xr_hbm, idx_hbm, out_hbm,
       srcv, dstv, msgs, wb, acc, gsem, ssem) = refs

    c = lax.axis_index("c")
    s = lax.axis_index("s")

    pltpu.async_copy(idx_hbm.at[c, s], srcv, gsem[0])
    pltpu.async_copy(idx_hbm.at[2, s], dstv, gsem[1])

    z16 = jnp.zeros((16,), jnp.float32)

    def zrow(r, carry):
      for k in range(DH // 16):
        wb[r, pl.ds(k * 16, 16)] = z16
      return carry

    lax.fori_loop(0, WB, zrow, 0)

    off = 0
    while off < ZROWS:
      n = min(WB, ZROWS - off)
      pltpu.sync_copy(wb.at[pl.ds(0, n)],
                      acc.at[pl.ds(s * ZROWS + off, n)])
      off += n

    if with_counts:
      for k in range(8):
        ones[pl.ds(k * 16, 16)] = jnp.ones((16,), jnp.float32)
      for k in range(63):
        zc[pl.ds(k * 16, 16)] = z16

      @pl.when((c == 0) & (s < CP_TILES))
      def _():
        pltpu.sync_copy(zc.at[pl.ds(0, ROWS_PER_CP)],
                        cacc.at[pl.ds(s * ROWS_PER_CP, ROWS_PER_CP)])

    pltpu.make_async_copy(idx_hbm.at[c, s], srcv, gsem[0]).wait()
    pltpu.make_async_copy(idx_hbm.at[2, s], dstv, gsem[1]).wait()
    plsc.subcore_barrier()

    do_counts = with_counts

    def step(i, carry):
      for b in range(NBUF):
        j = i * NBUF + b

        @pl.when(i > 0)
        def _():
          pltpu.make_async_copy(
              msgs[b], acc.at[dstv.at[j - NBUF]], ssem[b]).wait()

        pltpu.async_copy(xr_hbm.at[srcv.at[j]], msgs[b], gsem[b])

      for b in range(NBUF):
        j = i * NBUF + b
        pltpu.make_async_copy(xr_hbm.at[srcv.at[j]], msgs[b], gsem[b]).wait()
        pltpu.async_copy(msgs[b], acc.at[dstv.at[j]], ssem[b], add=True)
        if do_counts:
          @pl.when(c == 0)
          def _():
            pltpu.async_copy(ones.at[pl.ds(0, CHUNK)], cacc.at[dstv.at[j]],
                             csem, add=True)
      return carry

    lax.fori_loop(0, NR, step, 0)

    for b in range(NBUF):
      pltpu.make_async_copy(
          msgs[b], acc.at[dstv.at[CPT - NBUF + b]], ssem[b]).wait()
    if do_counts:
      @pl.when(c == 0)
      def _():
        def cdrain(j, carry):
          pltpu.make_async_copy(ones.at[pl.ds(0, CHUNK)],
                                cacc.at[dstv.at[j]], csem).wait()
          return carry
        lax.fori_loop(0, CPT, cdrain, 0)

    plsc.subcore_barrier()

    off = 0
    while off < WROWS:
      n = min(WB, WROWS - off)
      r0 = s * WROWS + off
      pltpu.sync_copy(acc.at[pl.ds(r0, n)], wb.at[pl.ds(0, n)])
      pltpu.sync_copy(wb.at[pl.ds(0, n)],
                      out_hbm.at[pl.ds(r0, n), pl.ds(c * DH, DH)])
      off += n

    if with_counts:
      @pl.when((c == 0) & (s < CP_TILES))
      def _():
        pltpu.sync_copy(cacc.at[pl.ds(s * ROWS_PER_CP, ROWS_PER_CP)],
                        zc.at[pl.ds(0, ROWS_PER_CP)])
        pltpu.sync_copy(zc.at[pl.ds(0, ROWS_PER_CP)],
                        cnt_hbm.at[pl.ds(s * ROWS_PER_CP, ROWS_PER_CP)])

  return pl.kernel(
      body, out_type=out_type, mesh=mesh, scratch_types=scratch,
      compiler_params=pltpu.CompilerParams(use_tc_tiling_on_sc=False))


_sc_agg_counts = _make_sc_agg(True)
_sc_agg = _make_sc_agg(False)

def _dense_body(p_ref, cnt_ref, x_ref, wl_ref, b_ref, wr_ref, o_ref):
  cnt = jnp.maximum(cnt_ref[...], 1.0)
  dn0 = (((0,), (0,)), ((), ()))
  cnt_col = lax.dot_general(cnt, jnp.ones((1, 1), jnp.float32), dn0,
                            preferred_element_type=jnp.float32,
                            precision=lax.Precision.HIGHEST)
  agg = p_ref[...] / cnt_col
  dn = (((1,), (1,)), ((), ()))
  h = lax.dot_general(agg, wl_ref[...], dn, preferred_element_type=jnp.float32)
  h = h + lax.dot_general(x_ref[...], wr_ref[...], dn,
                          preferred_element_type=jnp.float32)
  o_ref[...] = jnp.maximum(h + b_ref[...], 0.0)


_dense = pl.pallas_call(
    _dense_body,
    in_specs=[
        pl.BlockSpec((N_NODES, D), lambda: (0, 0)),
        pl.BlockSpec((1, N_NODES), lambda: (0, 0)),
        pl.BlockSpec((N_NODES, D), lambda: (0, 0)),
        pl.BlockSpec((D, D), lambda: (0, 0)),
        pl.BlockSpec((1, D), lambda: (0, 0)),
        pl.BlockSpec((D, D), lambda: (0, 0)),
    ],
    out_specs=pl.BlockSpec((N_NODES, D), lambda: (0, 0)),
    out_shape=jax.ShapeDtypeStruct((N_NODES, D), jnp.float32),
)


@jax.jit
def kernel(x, edge_index, W1_l, b1_l, W1_r, W2_l, b2_l, W2_r):
  ei = edge_index.astype(jnp.int32)
  s2 = ei[0:1] * 2
  idx = jnp.concatenate([s2, s2 + 1, ei[1:2]], axis=0).reshape(
      3, NS, CPT, CHUNK)
  xr = x.reshape(2 * N_NODES, DH)

  p1, cnt = _sc_agg_counts(xr, idx)
  cnt2 = cnt.reshape(1, N_NODES)
  h = _dense(p1, cnt2, x, W1_l, b1_l.reshape(1, D), W1_r)
  (p2,) = _sc_agg(h.reshape(2 * N_NODES, DH), idx)
  return _dense(p2, cnt2, h, W2_l, b2_l.reshape(1, D), W2_r)
